# Initial kernel scaffold; baseline (speedup 1.0000x reference)
import jax
import jax.numpy as jnp
from jax.experimental import pallas as pl


def kernel(system_states_trajectory, eligibility_traces_trajectory, inverse_state_norms_trajectory, variational_gradient_trajectory, post_gain, weight_values, trophic_support_map, weight_rows, weight_cols, active_blocks, pruning_threshold):
    raise NotImplementedError("write your pallas kernel here")


# speedup vs baseline: 1.7936x; 1.7936x over previous
import functools

import jax
import jax.numpy as jnp
from jax import lax
from jax.experimental import pallas as pl
from jax.experimental.pallas import tpu as pltpu
from jax.experimental.pallas import tpu_sc as plsc

EPS = 1e-8
NB = 128
NPB = 128
NS = 2048
T = 16
B = 16
N = NB * NPB
ALPHA = 0.01
MAX_NORM = 10.0
DELTA_MAX_NORM = 1.0
TBR = T * B

CHUNK = 2048
KSLOTS = 32


def _reduce_body(inv_ref, ss_ref, et_ref, vg_ref, pg_ref,
                 rf_ref, hb_ref, oj_ref, ax_ref, de_ref):
    inv = inv_ref[...]
    vg = vg_ref[...]
    et = et_ref[...]
    pg = pg_ref[...]
    ss = ss_ref[...]
    rf = jnp.sum(vg * inv, axis=0, keepdims=True)
    rf_ref[...] = rf
    hb_ref[...] = jnp.sum(et * pg * inv, axis=0, keepdims=True)
    oj_ref[...] = jnp.sum(ss * inv, axis=0, keepdims=True)
    ax_ref[...] = jnp.sum(et, axis=0, keepdims=True) * (1.0 / TBR)
    pgm = jnp.sum(pg, axis=0, keepdims=True) * (1.0 / TBR)
    de_ref[...] = rf * pgm


def _reduce_fields(inv2, ss2, et2, vg2, pg2):
    grid = (N // CHUNK,)
    big = pl.BlockSpec((TBR, CHUNK), lambda i: (0, i))
    out = pl.BlockSpec((1, CHUNK), lambda i: (0, i))
    return pl.pallas_call(
        _reduce_body,
        grid=grid,
        in_specs=[pl.BlockSpec((TBR, 1), lambda i: (0, 0)), big, big, big, big],
        out_specs=[out, out, out, out, out],
        out_shape=[jax.ShapeDtypeStruct((1, N), jnp.float32)] * 5,
    )(inv2, ss2, et2, vg2, pg2)


def _trophic_body(ax_ref, de_ref, tsm_ref, out_ref):
    ax = ax_ref[...]
    de = de_ref[...]
    ti = lax.dot_general(ax, de, (((1,), (1,)), ((), ())),
                         preferred_element_type=jnp.float32)
    ti = jnp.abs(ti) / jnp.float32(NPB + EPS)
    new = tsm_ref[...] * (1.0 - ALPHA) + ALPHA * ti
    r = lax.broadcasted_iota(jnp.int32, (NB, NB), 0)
    c = lax.broadcasted_iota(jnp.int32, (NB, NB), 1)
    out_ref[...] = jnp.where(r == c, 0.0, new)


def _trophic(ax_blk, de_blk, tsm):
    return pl.pallas_call(
        _trophic_body,
        out_shape=jax.ShapeDtypeStruct((NB, NB), jnp.float32),
    )(ax_blk, de_blk, tsm)


def _gather_rows(fb_blk, hb_blk, oj_blk, rows, cols):
    try:
        info = plsc.get_sparse_core_info()
        nc, nsub = info.num_cores, info.num_subcores
    except Exception:
        nc, nsub = 2, 16
    nw = nc * nsub
    bpw = NS // nw
    mesh = plsc.VectorSubcoreMesh(core_axis_name="c", subcore_axis_name="s")

    @functools.partial(
        pl.kernel,
        out_type=[jax.ShapeDtypeStruct((NS, NPB), jnp.float32)] * 3,
        mesh=mesh,
        scratch_types=[
            pltpu.VMEM((bpw,), jnp.int32),
            pltpu.VMEM((bpw,), jnp.int32),
            pltpu.VMEM((bpw, NPB), jnp.float32),
            pltpu.VMEM((bpw, NPB), jnp.float32),
            pltpu.VMEM((bpw, NPB), jnp.float32),
            pltpu.SemaphoreType.DMA,
        ],
    )
    def gk(fb_h, hb_h, oj_h, rows_h, cols_h, post_h, pre_h, y_h,
           ri_v, ci_v, post_v, pre_v, y_v, sem):
        wid = lax.axis_index("s") * nc + lax.axis_index("c")
        base = wid * bpw
        pltpu.sync_copy(rows_h.at[pl.ds(base, bpw)], ri_v)
        pltpu.sync_copy(cols_h.at[pl.ds(base, bpw)], ci_v)
        c1 = pltpu.async_copy(fb_h.at[ri_v], post_v, sem)
        c2 = pltpu.async_copy(hb_h.at[ci_v], pre_v, sem)
        c3 = pltpu.async_copy(oj_h.at[ri_v], y_v, sem)
        c1.wait()
        c2.wait()
        c3.wait()
        pltpu.sync_copy(post_v, post_h.at[pl.ds(base, bpw)])
        pltpu.sync_copy(pre_v, pre_h.at[pl.ds(base, bpw)])
        pltpu.sync_copy(y_v, y_h.at[pl.ds(base, bpw)])

    return gk(fb_blk, hb_blk, oj_blk, rows, cols)


def _update_body(thr_ref, act_ref, post_ref, pre_ref, y_ref, w_ref, out_ref):
    w = w_ref[...]
    post = post_ref[...][:, :, None]
    pre = pre_ref[...][:, None, :]
    y = y_ref[...]
    y2 = (y * y)[:, :, None]
    delta = post * pre - y2 * w
    dn = jnp.sqrt(jnp.sum(delta * delta, axis=(1, 2), keepdims=True))
    s1 = jnp.minimum(1.0, DELTA_MAX_NORM / (dn + EPS))
    wn = w + delta * s1
    thr = thr_ref[0]
    wn = jnp.where(jnp.abs(wn) < thr, 0.0, wn)
    wnorm = jnp.sqrt(jnp.sum(wn * wn, axis=(1, 2), keepdims=True))
    s2 = jnp.minimum(1.0, MAX_NORM / (wnorm + EPS))
    wn = wn * s2
    act = act_ref[...][:, :, None]
    out_ref[...] = jnp.where(act > 0, wn, w)


def _update_weights(thr_arr, actf, post, pre, y, w):
    grid = (NS // KSLOTS,)
    vec = pl.BlockSpec((KSLOTS, NPB), lambda i: (i, 0))
    return pl.pallas_call(
        _update_body,
        grid=grid,
        in_specs=[
            pl.BlockSpec(memory_space=pltpu.SMEM),
            pl.BlockSpec((KSLOTS, 1), lambda i: (i, 0)),
            vec, vec, vec,
            pl.BlockSpec((KSLOTS, NPB, NPB), lambda i: (i, 0, 0)),
        ],
        out_specs=pl.BlockSpec((KSLOTS, NPB, NPB), lambda i: (i, 0, 0)),
        out_shape=jax.ShapeDtypeStruct((NS, NPB, NPB), jnp.float32),
    )(thr_arr, actf, post, pre, y, w)


def kernel(system_states_trajectory, eligibility_traces_trajectory,
           inverse_state_norms_trajectory, variational_gradient_trajectory,
           post_gain, weight_values, trophic_support_map,
           weight_rows, weight_cols, active_blocks, pruning_threshold):
    ss2 = system_states_trajectory.reshape(TBR, N)
    et2 = eligibility_traces_trajectory.reshape(TBR, N)
    vg2 = variational_gradient_trajectory.reshape(TBR, N)
    pg2 = post_gain.reshape(TBR, N)
    inv2 = inverse_state_norms_trajectory.reshape(TBR, 1)

    rf, hb, oj, axf, den = _reduce_fields(inv2, ss2, et2, vg2, pg2)

    tsm_new = _trophic(axf.reshape(NB, NPB), den.reshape(NB, NPB),
                       trophic_support_map)

    post, pre, y = _gather_rows(
        rf.reshape(NB, NPB), hb.reshape(NB, NPB), oj.reshape(NB, NPB),
        weight_rows.astype(jnp.int32), weight_cols.astype(jnp.int32))

    thr_arr = jnp.asarray(pruning_threshold, jnp.float32).reshape(1)
    actf = active_blocks.astype(jnp.float32).reshape(NS, 1)
    w_out = _update_weights(thr_arr, actf, post, pre, y, weight_values)
    return (w_out, tsm_new)
